# baseline (device time: 69222 ns/iter reference)
import jax
import jax.numpy as jnp
from jax import lax
from jax.experimental import pallas as pl
from jax.experimental.pallas import tpu as pltpu

N_DEV = 4
B, Sq, D = 2, 256, 512
Hq, Dh = 16, 64
H_LOC = Hq // N_DEV
HD_LOC = H_LOC * Dh
SKV_LOC = 256
SKV = SKV_LOC * N_DEV
DOUT = 512
BLK = 64
SCALE = 0.125
NEG = -1e9

_CompilerParams = getattr(pltpu, "CompilerParams", None) or pltpu.TPUCompilerParams


def kernel(x, Wq, K_ext, V_ext, Wo):
    def body(x_ref, wq_ref, k_ref, v_ref, wo_ref, out_ref,
             k_flat, v_flat, k_all, v_all, my_part, part_gather,
             send_k, send_v, send_p, recv_k, recv_v, recv_p):
        me = lax.axis_index("i")

        barrier = pltpu.get_barrier_semaphore()
        for d in range(1, N_DEV):
            pl.semaphore_signal(
                barrier, inc=1,
                device_id=((me + d) % N_DEV,),
                device_id_type=pl.DeviceIdType.MESH,
            )
        pl.semaphore_wait(barrier, N_DEV - 1)

        k_flat[...] = k_ref[...].reshape(B, SKV_LOC, Hq * Dh)
        v_flat[...] = v_ref[...].reshape(B, SKV_LOC, Hq * Dh)

        kv_rdmas = []
        for d in range(1, N_DEV):
            peer = (me + d) % N_DEV
            slot = N_DEV - 1 - d
            for (flat, gathered, ssem, rsem) in (
                (k_flat, k_all, send_k, recv_k),
                (v_flat, v_all, send_v, recv_v),
            ):
                rdma = pltpu.make_async_remote_copy(
                    src_ref=flat.at[:, :, pl.ds(peer * HD_LOC, HD_LOC)],
                    dst_ref=gathered.at[:, pl.ds(me * SKV_LOC, SKV_LOC), :],
                    send_sem=ssem.at[d - 1],
                    recv_sem=rsem.at[slot],
                    device_id=(peer,),
                    device_id_type=pl.DeviceIdType.MESH,
                )
                rdma.start()
                kv_rdmas.append(rdma)

        k_all[:, pl.ds(me * SKV_LOC, SKV_LOC), :] = (
            k_flat[:, :, pl.ds(me * HD_LOC, HD_LOC)])
        v_all[:, pl.ds(me * SKV_LOC, SKV_LOC), :] = (
            v_flat[:, :, pl.ds(me * HD_LOC, HD_LOC)])

        q = jnp.dot(x_ref[...].reshape(B * Sq, D), wq_ref[...],
                    preferred_element_type=jnp.float32)

        for d in range(1, N_DEV):
            src = (me + d) % N_DEV
            for (gathered, rsem) in ((k_all, recv_k), (v_all, recv_v)):
                pltpu.make_async_remote_copy(
                    src_ref=gathered.at[:, pl.ds(src * SKV_LOC, SKV_LOC), :],
                    dst_ref=gathered.at[:, pl.ds(src * SKV_LOC, SKV_LOC), :],
                    send_sem=rsem.at[d - 1],
                    recv_sem=rsem.at[d - 1],
                    device_id=(src,),
                    device_id_type=pl.DeviceIdType.MESH,
                ).wait_recv()

        qb = lax.broadcasted_iota(jnp.int32, (Sq, SKV), 0) // BLK
        kb = lax.broadcasted_iota(jnp.int32, (Sq, SKV), 1) // BLK
        mask = (qb == kb) | (kb == 0) | (((qb + kb) % 3) == 0)

        parts = []
        for b in range(B):
            kb_all = k_all[b]
            vb_all = v_all[b]
            ctx_h = []
            for h in range(H_LOC):
                qbh = lax.slice(q, (b * Sq, h * Dh), ((b + 1) * Sq, (h + 1) * Dh))
                kbh = lax.slice(kb_all, (0, h * Dh), (SKV, (h + 1) * Dh))
                vbh = lax.slice(vb_all, (0, h * Dh), (SKV, (h + 1) * Dh))
                s = lax.dot_general(qbh, kbh, (((1,), (1,)), ((), ())),
                                    preferred_element_type=jnp.float32) * SCALE
                s = jnp.where(mask, s, jnp.float32(NEG))
                m = jnp.max(s, axis=1, keepdims=True)
                w = jnp.exp(s - m)
                w = w / jnp.sum(w, axis=1, keepdims=True)
                ctx_h.append(lax.dot_general(w, vbh, (((1,), (0,)), ((), ())),
                                             preferred_element_type=jnp.float32))
            ctx_b = jnp.concatenate(ctx_h, axis=1)
            parts.append(jnp.dot(ctx_b, wo_ref[...],
                                 preferred_element_type=jnp.float32))
        my_part[...] = jnp.stack(parts)

        p_rdmas = []
        for d in range(1, N_DEV):
            peer = (me + d) % N_DEV
            slot = N_DEV - 1 - d
            rdma = pltpu.make_async_remote_copy(
                src_ref=my_part,
                dst_ref=part_gather.at[slot],
                send_sem=send_p.at[d - 1],
                recv_sem=recv_p.at[slot],
                device_id=(peer,),
                device_id_type=pl.DeviceIdType.MESH,
            )
            rdma.start()
            p_rdmas.append(rdma)

        for d in range(1, N_DEV):
            pltpu.make_async_remote_copy(
                src_ref=my_part,
                dst_ref=part_gather.at[d - 1],
                send_sem=recv_p.at[d - 1],
                recv_sem=recv_p.at[d - 1],
                device_id=((me + d) % N_DEV,),
                device_id_type=pl.DeviceIdType.MESH,
            ).wait_recv()

        acc = my_part[...]
        for j in range(N_DEV - 1):
            acc = acc + part_gather[j]
        out_ref[...] = acc

        for rdma in kv_rdmas + p_rdmas:
            rdma.wait_send()

    return pl.pallas_call(
        body,
        out_shape=jax.ShapeDtypeStruct((B, Sq, DOUT), jnp.float32),
        in_specs=[pl.BlockSpec(memory_space=pltpu.VMEM)] * 5,
        out_specs=pl.BlockSpec(memory_space=pltpu.VMEM),
        scratch_shapes=[
            pltpu.VMEM((B, SKV_LOC, Hq * Dh), jnp.float32),
            pltpu.VMEM((B, SKV_LOC, Hq * Dh), jnp.float32),
            pltpu.VMEM((B, SKV, HD_LOC), jnp.float32),
            pltpu.VMEM((B, SKV, HD_LOC), jnp.float32),
            pltpu.VMEM((B, Sq, DOUT), jnp.float32),
            pltpu.VMEM((N_DEV - 1, B, Sq, DOUT), jnp.float32),
            pltpu.SemaphoreType.DMA((N_DEV - 1,)),
            pltpu.SemaphoreType.DMA((N_DEV - 1,)),
            pltpu.SemaphoreType.DMA((N_DEV - 1,)),
            pltpu.SemaphoreType.DMA((N_DEV - 1,)),
            pltpu.SemaphoreType.DMA((N_DEV - 1,)),
            pltpu.SemaphoreType.DMA((N_DEV - 1,)),
        ],
        compiler_params=_CompilerParams(collective_id=0),
    )(x, Wq, K_ext, V_ext, Wo)
